# Initial kernel scaffold; baseline (speedup 1.0000x reference)
import jax
import jax.numpy as jnp
from jax.experimental import pallas as pl


def kernel(d, x, ss, vb, npatches, patches, npp, sRef, A, Tarr, meanY, ds, lam2, device):
    raise NotImplementedError("write your pallas kernel here")


# speedup vs baseline: 6.2582x; 6.2582x over previous
import functools

import jax
import jax.numpy as jnp
from jax import lax
from jax.experimental import pallas as pl
from jax.experimental.pallas import tpu as pltpu
from jax.experimental.pallas import tpu_sc as plsc

N = 65536
P = 64
NP = 16384
K = 256
M = 1024

NW = 32
PAIRS = P * NP
PAIRS_PER_W = PAIRS // NW
SC_CHUNK = 8192

BN_SSB = 2048
BN_MV = 2048


def _ssb_body(x_ref, d_ref, meanyt_ref, out_ref):
    prod = lax.dot_general(
        x_ref[...], d_ref[...],
        dimension_numbers=(((0,), (1,)), ((), ())),
        preferred_element_type=jnp.float32)
    out_ref[...] = prod + meanyt_ref[...]


def _ssb_transposed(x, d, meany_t):
    grid = NP // BN_SSB
    return pl.pallas_call(
        _ssb_body,
        grid=(grid,),
        in_specs=[
            pl.BlockSpec((K, BN_SSB), lambda j: (0, j)),
            pl.BlockSpec((P, K), lambda j: (0, 0)),
            pl.BlockSpec((BN_SSB, 1), lambda j: (j, 0)),
        ],
        out_specs=pl.BlockSpec((BN_SSB, P), lambda j: (j, 0)),
        out_shape=jax.ShapeDtypeStruct((NP, P), jnp.float32),
    )(x, d, meany_t)


def _sc_scatter_body(idx_hbm, val_hbm, out_hbm, acc, idxv, valv):
    wid = lax.axis_index("s") * 2 + lax.axis_index("c")
    base = wid * PAIRS_PER_W

    def zero_body(i, carry):
        acc[pl.ds(i * 16, 16)] = jnp.zeros((16,), jnp.float32)
        return carry
    lax.fori_loop(0, N // 16, zero_body, 0)

    def chunk_body(c, carry):
        off = base + c * SC_CHUNK
        pltpu.sync_copy(idx_hbm.at[pl.ds(off, SC_CHUNK)], idxv)
        pltpu.sync_copy(val_hbm.at[pl.ds(off, SC_CHUNK)], valv)

        def inner(j, icarry):
            i16 = idxv[pl.ds(j * 16, 16)]
            v16 = valv[pl.ds(j * 16, 16)]
            plsc.addupdate_scatter(acc, [i16], v16)
            return icarry
        lax.fori_loop(0, SC_CHUNK // 16, inner, carry)
        return carry
    lax.fori_loop(0, PAIRS_PER_W // SC_CHUNK, chunk_body, 0)

    pltpu.sync_copy(acc, out_hbm.at[wid])


@functools.cache
def _get_sc_scatter():
    return functools.partial(
        pl.kernel,
        out_type=jax.ShapeDtypeStruct((NW, N), jnp.float32),
        mesh=plsc.VectorSubcoreMesh(core_axis_name="c", subcore_axis_name="s"),
        scratch_types=[
            pltpu.VMEM((N,), jnp.float32),
            pltpu.VMEM((SC_CHUNK,), jnp.int32),
            pltpu.VMEM((SC_CHUNK,), jnp.float32),
        ],
        compiler_params=pltpu.CompilerParams(needs_layout_passes=False),
    )(_sc_scatter_body)


def _mv_body(lam2_ref, part_ref, ds_ref, npp_ref, vb_ref, sref_ref,
             a_ref, tarr_ref, out_ref, acc_ref):
    j = pl.program_id(0)

    @pl.when(j == 0)
    def _init():
        acc_ref[...] = jnp.zeros_like(acc_ref)

    lam2 = lam2_ref[0, 0]
    ssp = jnp.sum(part_ref[...], axis=0, keepdims=True)
    v = ((lam2 * ds_ref[...] + ssp) / (lam2 + npp_ref[...])) \
        * vb_ref[...] + sref_ref[...]
    acc_ref[...] += jnp.sum(a_ref[...] * v, axis=1, keepdims=True)

    @pl.when(j == pl.num_programs(0) - 1)
    def _fin():
        diff = acc_ref[...] - tarr_ref[...]
        out_ref[...] = jnp.mean(diff * diff).reshape(1, 1)


def _matvec_mse(lam2, partials, ds_r, npp_r, vb_r, sref_r, a, tarr):
    grid = N // BN_MV
    return pl.pallas_call(
        _mv_body,
        grid=(grid,),
        in_specs=[
            pl.BlockSpec((1, 1), lambda j: (0, 0)),
            pl.BlockSpec((NW, BN_MV), lambda j: (0, j)),
            pl.BlockSpec((1, BN_MV), lambda j: (0, j)),
            pl.BlockSpec((1, BN_MV), lambda j: (0, j)),
            pl.BlockSpec((1, BN_MV), lambda j: (0, j)),
            pl.BlockSpec((1, BN_MV), lambda j: (0, j)),
            pl.BlockSpec((M, BN_MV), lambda j: (0, j)),
            pl.BlockSpec((M, 1), lambda j: (0, 0)),
        ],
        out_specs=pl.BlockSpec((1, 1), lambda j: (0, 0)),
        out_shape=jax.ShapeDtypeStruct((1, 1), jnp.float32),
        scratch_shapes=[pltpu.VMEM((M, 1), jnp.float32)],
        compiler_params=pltpu.CompilerParams(
            dimension_semantics=("arbitrary",)),
    )(lam2, partials, ds_r, npp_r, vb_r, sref_r, a, tarr)


def kernel(d, x, ss, vb, npatches, patches, npp, sRef, A, Tarr, meanY,
           ds, lam2, device):
    meany_t = meanY.reshape(NP, 1)
    ss_bt = _ssb_transposed(x, d, meany_t)
    idx_t = patches.T.reshape(-1)
    partials = _get_sc_scatter()(idx_t, ss_bt.reshape(-1))
    loss = _matvec_mse(
        lam2.reshape(1, 1), partials,
        ds.reshape(1, N), npp.reshape(1, N), vb.reshape(1, N),
        sRef.reshape(1, N), A, Tarr)
    return loss[0, 0]


# speedup vs baseline: 7.0783x; 1.1311x over previous
import functools

import jax
import jax.numpy as jnp
from jax import lax
from jax.experimental import pallas as pl
from jax.experimental.pallas import tpu as pltpu
from jax.experimental.pallas import tpu_sc as plsc

N = 65536
P = 64
NP = 16384
K = 256
M = 1024

NW = 32
PAIRS = P * NP
PAIRS_PER_W = PAIRS // NW
SC_CHUNK = 8192

BN_SSB = 2048
BN_MV = 2048


def _ssb_body(x_ref, d_ref, meanyt_ref, out_ref):
    prod = lax.dot_general(
        x_ref[...], d_ref[...],
        dimension_numbers=(((0,), (1,)), ((), ())),
        preferred_element_type=jnp.float32)
    out_ref[...] = prod + meanyt_ref[...]


def _ssb_transposed(x, d, meany_t):
    grid = NP // BN_SSB
    return pl.pallas_call(
        _ssb_body,
        grid=(grid,),
        in_specs=[
            pl.BlockSpec((K, BN_SSB), lambda j: (0, j)),
            pl.BlockSpec((P, K), lambda j: (0, 0)),
            pl.BlockSpec((BN_SSB, 1), lambda j: (j, 0)),
        ],
        out_specs=pl.BlockSpec((BN_SSB, P), lambda j: (j, 0)),
        out_shape=jax.ShapeDtypeStruct((NP, P), jnp.float32),
    )(x, d, meany_t)


_NCH = PAIRS_PER_W // SC_CHUNK
_UNROLL = 8


def _sc_scatter_body(idx_hbm, val_hbm, out_hbm, acc, idxv, valv,
                     s0, s1, s2, s3):
    wid = lax.axis_index("s") * 2 + lax.axis_index("c")
    base = wid * PAIRS_PER_W
    sems = (s0, s1, s2, s3)

    copies = [None, None]

    def start(c, b):
        off = base + c * SC_CHUNK
        h1 = pltpu.async_copy(idx_hbm.at[pl.ds(off, SC_CHUNK)],
                              idxv.at[b], sems[2 * b])
        h2 = pltpu.async_copy(val_hbm.at[pl.ds(off, SC_CHUNK)],
                              valv.at[b], sems[2 * b + 1])
        copies[b] = (h1, h2)

    start(0, 0)

    z16 = jnp.zeros((16,), jnp.float32)

    def zero_body(i, carry):
        for u in range(_UNROLL):
            acc[pl.ds((i * _UNROLL + u) * 16, 16)] = z16
        return carry
    lax.fori_loop(0, N // 16 // _UNROLL, zero_body, 0)

    for c in range(_NCH):
        b = c % 2
        if c + 1 < _NCH:
            start(c + 1, 1 - b)
        copies[b][0].wait()
        copies[b][1].wait()

        def inner(j, icarry):
            for u in range(_UNROLL):
                s = (j * _UNROLL + u) * 16
                i16 = idxv[b, pl.ds(s, 16)]
                v16 = valv[b, pl.ds(s, 16)]
                plsc.addupdate_scatter(acc, [i16], v16)
            return icarry
        lax.fori_loop(0, SC_CHUNK // 16 // _UNROLL, inner, 0)

    pltpu.sync_copy(acc, out_hbm.at[wid])


@functools.cache
def _get_sc_scatter():
    return functools.partial(
        pl.kernel,
        out_type=jax.ShapeDtypeStruct((NW, N), jnp.float32),
        mesh=plsc.VectorSubcoreMesh(core_axis_name="c", subcore_axis_name="s"),
        scratch_types=[
            pltpu.VMEM((N,), jnp.float32),
            pltpu.VMEM((2, SC_CHUNK), jnp.int32),
            pltpu.VMEM((2, SC_CHUNK), jnp.float32),
            pltpu.SemaphoreType.DMA,
            pltpu.SemaphoreType.DMA,
            pltpu.SemaphoreType.DMA,
            pltpu.SemaphoreType.DMA,
        ],
        compiler_params=pltpu.CompilerParams(needs_layout_passes=False),
    )(_sc_scatter_body)


def _mv_body(lam2_ref, part_ref, ds_ref, npp_ref, vb_ref, sref_ref,
             a_ref, tarr_ref, out_ref, acc_ref):
    j = pl.program_id(0)

    @pl.when(j == 0)
    def _init():
        acc_ref[...] = jnp.zeros_like(acc_ref)

    lam2 = lam2_ref[0, 0]
    ssp = jnp.sum(part_ref[...], axis=0, keepdims=True)
    v = ((lam2 * ds_ref[...] + ssp) / (lam2 + npp_ref[...])) \
        * vb_ref[...] + sref_ref[...]
    acc_ref[...] += jnp.sum(a_ref[...] * v, axis=1, keepdims=True)

    @pl.when(j == pl.num_programs(0) - 1)
    def _fin():
        diff = acc_ref[...] - tarr_ref[...]
        out_ref[...] = jnp.mean(diff * diff).reshape(1, 1)


def _matvec_mse(lam2, partials, ds_r, npp_r, vb_r, sref_r, a, tarr):
    grid = N // BN_MV
    return pl.pallas_call(
        _mv_body,
        grid=(grid,),
        in_specs=[
            pl.BlockSpec((1, 1), lambda j: (0, 0)),
            pl.BlockSpec((NW, BN_MV), lambda j: (0, j)),
            pl.BlockSpec((1, BN_MV), lambda j: (0, j)),
            pl.BlockSpec((1, BN_MV), lambda j: (0, j)),
            pl.BlockSpec((1, BN_MV), lambda j: (0, j)),
            pl.BlockSpec((1, BN_MV), lambda j: (0, j)),
            pl.BlockSpec((M, BN_MV), lambda j: (0, j)),
            pl.BlockSpec((M, 1), lambda j: (0, 0)),
        ],
        out_specs=pl.BlockSpec((1, 1), lambda j: (0, 0)),
        out_shape=jax.ShapeDtypeStruct((1, 1), jnp.float32),
        scratch_shapes=[pltpu.VMEM((M, 1), jnp.float32)],
        compiler_params=pltpu.CompilerParams(
            dimension_semantics=("arbitrary",)),
    )(lam2, partials, ds_r, npp_r, vb_r, sref_r, a, tarr)


def kernel(d, x, ss, vb, npatches, patches, npp, sRef, A, Tarr, meanY,
           ds, lam2, device):
    meany_t = meanY.reshape(NP, 1)
    ss_bt = _ssb_transposed(x, d, meany_t)
    idx_t = patches.T.reshape(-1)
    partials = _get_sc_scatter()(idx_t, ss_bt.reshape(-1))
    loss = _matvec_mse(
        lam2.reshape(1, 1), partials,
        ds.reshape(1, N), npp.reshape(1, N), vb.reshape(1, N),
        sRef.reshape(1, N), A, Tarr)
    return loss[0, 0]


# speedup vs baseline: 8.2411x; 1.1643x over previous
import functools

import jax
import jax.numpy as jnp
from jax import lax
from jax.experimental import pallas as pl
from jax.experimental.pallas import tpu as pltpu
from jax.experimental.pallas import tpu_sc as plsc

N = 65536
P = 64
NP = 16384
K = 256
M = 1024

NW = 32
PAIRS = P * NP
PAIRS_PER_W = PAIRS // NW
SC_CHUNK = 8192

BN_SSB = 2048
BN_MV = 4096


def _ssb_body(xl_ref, xr_ref, d_ref, myl_ref, myr_ref, out_ref):
    dn = (((0,), (1,)), ((), ()))
    left = lax.dot_general(xl_ref[...], d_ref[...], dimension_numbers=dn,
                           preferred_element_type=jnp.float32) \
        + jnp.transpose(myl_ref[...])
    right = lax.dot_general(xr_ref[...], d_ref[...], dimension_numbers=dn,
                            preferred_element_type=jnp.float32) \
        + jnp.transpose(myr_ref[...])
    out_ref[...] = jnp.concatenate([left, right], axis=1)


def _ssb_transposed(x, d, meany):
    half = NP // (2 * BN_SSB)
    return pl.pallas_call(
        _ssb_body,
        grid=(half,),
        in_specs=[
            pl.BlockSpec((K, BN_SSB), lambda j: (0, j)),
            pl.BlockSpec((K, BN_SSB), lambda j: (0, j + half)),
            pl.BlockSpec((P, K), lambda j: (0, 0)),
            pl.BlockSpec((1, BN_SSB), lambda j: (0, j)),
            pl.BlockSpec((1, BN_SSB), lambda j: (0, j + half)),
        ],
        out_specs=pl.BlockSpec((BN_SSB, 2 * P), lambda j: (j, 0)),
        out_shape=jax.ShapeDtypeStruct((NP // 2, 2 * P), jnp.float32),
    )(x, x, d, meany, meany)


_NCH = PAIRS_PER_W // SC_CHUNK
_UNROLL = 8
ROWS_PER_W = (NP // 2) // NW
RW = ROWS_PER_W // _NCH


def _sc_scatter_body(idx_hbm, val_hbm, out_hbm, acc, idxv, valv,
                     s0, s1, s2, s3):
    wid = lax.axis_index("s") * 2 + lax.axis_index("c")
    r0 = wid * ROWS_PER_W
    val_sems = (s0, s1)
    idx_sems = (s2, s3)

    vcopies = [None, None]
    icopies = [None, None]

    def start_val(c, b):
        col = r0 + c * RW
        vcopies[b] = pltpu.async_copy(
            val_hbm.at[pl.ds(col * 2 * P, RW * 2 * P)],
            valv.at[b], val_sems[b])

    def start_idx(s, b):
        col = r0 + s * 2 * RW
        h1 = pltpu.async_copy(idx_hbm.at[:, pl.ds(col, 2 * RW)],
                              idxv.at[b, 0], idx_sems[b])
        h2 = pltpu.async_copy(idx_hbm.at[:, pl.ds(NP // 2 + col, 2 * RW)],
                              idxv.at[b, 1], idx_sems[b])
        icopies[b] = (h1, h2)

    start_idx(0, 0)
    start_val(0, 0)

    z16 = jnp.zeros((16,), jnp.float32)

    def zero_body(i, carry):
        for u in range(16):
            acc[pl.ds((i * 16 + u) * 16, 16)] = z16
        return carry
    lax.fori_loop(0, N // 16 // 16, zero_body, 0)

    iota16 = lax.iota(jnp.int32, 16)

    for c in range(_NCH):
        bv = c % 2
        bi = (c // 2) % 2
        if c + 1 < _NCH:
            start_val(c + 1, 1 - bv)
        if c % 2 == 0 and c + 2 < _NCH:
            start_idx((c + 2) // 2, 1 - bi)
        vcopies[bv].wait()
        if c % 2 == 0:
            for h in icopies[bi]:
                h.wait()
        colbase = (c % 2) * RW

        def inner(j, icarry):
            cols = jnp.full((16,), j + colbase, jnp.int32)
            groups = []
            for u in range(_UNROLL):
                rows = iota16 + (u % 4) * 16
                i16 = plsc.load_gather(idxv.at[bi, u // 4], [rows, cols])
                v16 = valv[bv, pl.ds(j * 2 * P + u * 16, 16)]
                groups.append((i16, v16))
            for i16, v16 in groups:
                plsc.addupdate_scatter(acc, [i16], v16)
            return icarry
        lax.fori_loop(0, RW, inner, 0)

    pltpu.sync_copy(acc, out_hbm.at[wid])


@functools.cache
def _get_sc_scatter():
    return functools.partial(
        pl.kernel,
        out_type=jax.ShapeDtypeStruct((NW, N), jnp.float32),
        mesh=plsc.VectorSubcoreMesh(core_axis_name="c", subcore_axis_name="s"),
        scratch_types=[
            pltpu.VMEM((N,), jnp.float32),
            pltpu.VMEM((2, 2, P, 2 * RW), jnp.int32),
            pltpu.VMEM((2, RW * 2 * P), jnp.float32),
            pltpu.SemaphoreType.DMA,
            pltpu.SemaphoreType.DMA,
            pltpu.SemaphoreType.DMA,
            pltpu.SemaphoreType.DMA,
        ],
        compiler_params=pltpu.CompilerParams(needs_layout_passes=False),
    )(_sc_scatter_body)


def _mv_body(lam2_ref, part_ref, ds_ref, npp_ref, vb_ref, sref_ref,
             a_ref, tarr_ref, out_ref, acc_ref):
    j = pl.program_id(0)

    @pl.when(j == 0)
    def _init():
        acc_ref[...] = jnp.zeros_like(acc_ref)

    lam2 = lam2_ref[0, 0]
    ssp = jnp.sum(part_ref[...], axis=0, keepdims=True)
    v = ((lam2 * ds_ref[...] + ssp) / (lam2 + npp_ref[...])) \
        * vb_ref[...] + sref_ref[...]
    acc_ref[...] += jnp.sum(a_ref[...] * v, axis=1, keepdims=True)

    @pl.when(j == pl.num_programs(0) - 1)
    def _fin():
        diff = acc_ref[...] - jnp.transpose(tarr_ref[...])
        out_ref[...] = jnp.mean(diff * diff).reshape(1, 1)


def _matvec_mse(lam2, partials, ds_r, npp_r, vb_r, sref_r, a, tarr):
    grid = N // BN_MV
    return pl.pallas_call(
        _mv_body,
        grid=(grid,),
        in_specs=[
            pl.BlockSpec((1, 1), lambda j: (0, 0)),
            pl.BlockSpec((NW, BN_MV), lambda j: (0, j)),
            pl.BlockSpec((1, BN_MV), lambda j: (0, j)),
            pl.BlockSpec((1, BN_MV), lambda j: (0, j)),
            pl.BlockSpec((1, BN_MV), lambda j: (0, j)),
            pl.BlockSpec((1, BN_MV), lambda j: (0, j)),
            pl.BlockSpec((M, BN_MV), lambda j: (0, j)),
            pl.BlockSpec((1, M), lambda j: (0, 0)),
        ],
        out_specs=pl.BlockSpec((1, 1), lambda j: (0, 0)),
        out_shape=jax.ShapeDtypeStruct((1, 1), jnp.float32),
        scratch_shapes=[pltpu.VMEM((M, 1), jnp.float32)],
        compiler_params=pltpu.CompilerParams(
            dimension_semantics=("arbitrary",)),
    )(lam2, partials, ds_r, npp_r, vb_r, sref_r, a, tarr)


def kernel(d, x, ss, vb, npatches, patches, npp, sRef, A, Tarr, meanY,
           ds, lam2, device):
    ss_bt = _ssb_transposed(x, d, meanY)
    partials = _get_sc_scatter()(patches, ss_bt.reshape(-1))
    loss = _matvec_mse(
        lam2.reshape(1, 1), partials,
        ds.reshape(1, N), npp.reshape(1, N), vb.reshape(1, N),
        sRef.reshape(1, N), A, Tarr.reshape(1, M))
    return loss[0, 0]
